# baseline (speedup 1.0000x reference)
import functools

import jax
import jax.numpy as jnp
from jax import lax
from jax.experimental import pallas as pl
from jax.experimental.pallas import tpu as pltpu
from jax.experimental.pallas import tpu_sc as plsc

N = 10000
D = 128
OUT_C = 12
N_OUT = 2000

N_PAD = 10240
NH = 5120
ACC_H = 5248
JUNK = NH
E = 320000
CH = 128
TILES = 32
E_PAD = 327680
EPT = E_PAD // TILES
NCH = EPT // CH
TPC = 16
RPT_A = ACC_H // TPC

_sc_cache = {}


def _build_sc_kernel():
    if _sc_cache:
        return _sc_cache["scat"]
    mesh = plsc.VectorSubcoreMesh(core_axis_name="c", subcore_axis_name="s")

    NBUF = 4
    ROUNDS = NCH // NBUF

    @functools.partial(
        pl.kernel,
        out_type=jax.ShapeDtypeStruct((4, ACC_H, D), jnp.float32),
        mesh=mesh,
        scratch_types=[
            pltpu.VMEM((NCH, CH), jnp.int32),
            pltpu.VMEM((NCH, CH), jnp.int32),
        ] + [pltpu.VMEM((CH, D), jnp.float32)] * NBUF + [
            pltpu.VMEM_SHARED((ACC_H, D), jnp.float32),
        ] + [pltpu.SemaphoreType.DMA] * (2 * NBUF),
    )
    def sc_scatter(g_hbm, src_hbm, dst_hbm, zeros_hbm, out_hbm,
                   src_v, dst_v, *rest):
        rows = rest[:NBUF]
        acc = rest[NBUF]
        sems = rest[NBUF + 1:]
        gsems = sems[:NBUF]
        ssems = sems[NBUF:]
        c = lax.axis_index("c")
        s = lax.axis_index("s")
        tile = c * TPC + s

        for q in (0, 1):
            base = q * (TILES * NCH) + tile * NCH
            pltpu.sync_copy(src_hbm.at[pl.ds(base, NCH)], src_v)
            pltpu.sync_copy(dst_hbm.at[pl.ds(base, NCH)], dst_v)
            pltpu.sync_copy(zeros_hbm,
                            acc.at[pl.ds(s * RPT_A, RPT_A)])
            plsc.subcore_barrier()

            def g_desc(k, b):
                return pltpu.make_async_copy(
                    g_hbm.at[src_v.at[k]], rows[b], gsems[b])

            def s_desc(k, b):
                return pltpu.make_async_copy(
                    rows[b], acc.at[dst_v.at[k]], ssems[b])

            for b in range(NBUF):
                g_desc(b, b).start()

            def round_body(r, carry):
                k0 = r * NBUF
                for b in range(NBUF):
                    g_desc(k0 + b, b).wait()
                    s_desc(k0 + b, b).start(add=True)
                for b in range(NBUF):
                    s_desc(k0 + b, b).wait()
                    g_desc(k0 + NBUF + b, b).start()
                return carry

            lax.fori_loop(0, ROUNDS - 1, round_body, 0)
            k0 = (ROUNDS - 1) * NBUF
            for b in range(NBUF):
                g_desc(k0 + b, b).wait()
                s_desc(k0 + b, b).start(add=True)
            for b in range(NBUF):
                s_desc(k0 + b, b).wait()
            plsc.subcore_barrier()
            pltpu.sync_copy(acc.at[pl.ds(s * RPT_A, RPT_A)],
                            out_hbm.at[2 * q + c].at[pl.ds(s * RPT_A, RPT_A)])

    _sc_cache["scat"] = sc_scatter
    return sc_scatter


def _sc_scatter(g, src2, dst2, zeros_big):
    return _build_sc_kernel()(g, src2, dst2, zeros_big)


_BLK = 1280
_GRID = N_PAD // _BLK


def _s_specs():
    return [
        pl.BlockSpec((1, _BLK, D), lambda i: (2 * (i // 4), i % 4, 0)),
        pl.BlockSpec((1, _BLK, D), lambda i: (2 * (i // 4) + 1, i % 4, 0)),
    ]


def _tc_layer1_body(x_ref, w_ref, d0_ref, d1_ref, g_ref, dis_ref):
    i = pl.program_id(0)
    rows = i * _BLK + lax.broadcasted_iota(jnp.int32, (_BLK, 1), 0)
    cnt = d0_ref[0, :, 0:1] + d1_ref[0, :, 0:1] + 1.0
    dis = lax.rsqrt(cnt) * (rows < N).astype(jnp.float32)
    dis_ref[...] = jnp.broadcast_to(dis, (_BLK, 16))
    h = jnp.dot(x_ref[...], w_ref[...], preferred_element_type=jnp.float32)
    g_ref[...] = h * dis


def _tc_layer1(x_pad, w1, deg4):
    return pl.pallas_call(
        _tc_layer1_body,
        grid=(_GRID,),
        in_specs=[
            pl.BlockSpec((_BLK, D), lambda i: (i, 0)),
            pl.BlockSpec((D, D), lambda i: (0, 0)),
        ] + _s_specs(),
        out_specs=[
            pl.BlockSpec((_BLK, D), lambda i: (i, 0)),
            pl.BlockSpec((_BLK, 16), lambda i: (i, 0)),
        ],
        out_shape=[
            jax.ShapeDtypeStruct((N_PAD, D), jnp.float32),
            jax.ShapeDtypeStruct((N_PAD, 16), jnp.float32),
        ],
    )(x_pad, w1, deg4, deg4)


def _tc_mid_body(s0_ref, s1_ref, g_ref, dis_ref, b_ref, w_ref, out_ref):
    d = dis_ref[:, 0:1]
    xk = (s0_ref[0] + s1_ref[0] + g_ref[...]) * d + b_ref[0]
    xk = jnp.maximum(xk, 0.0)
    h = jnp.dot(xk, w_ref[...], preferred_element_type=jnp.float32)
    out_ref[...] = h * d


def _tc_mid(s4, g_prev, dis16, b_prev, w_next):
    return pl.pallas_call(
        _tc_mid_body,
        grid=(_GRID,),
        in_specs=_s_specs() + [
            pl.BlockSpec((_BLK, D), lambda i: (i, 0)),
            pl.BlockSpec((_BLK, 16), lambda i: (i, 0)),
            pl.BlockSpec((1, D), lambda i: (0, 0)),
            pl.BlockSpec((D, D), lambda i: (0, 0)),
        ],
        out_specs=pl.BlockSpec((_BLK, D), lambda i: (i, 0)),
        out_shape=jax.ShapeDtypeStruct((N_PAD, D), jnp.float32),
    )(s4, s4, g_prev, dis16, b_prev, w_next)


def _tc_final_body(s0_ref, s1_ref, g_ref, dis_ref, b_ref, wout_ref, bout_ref,
                   out_ref):
    d = dis_ref[:, 0:1]
    s = s0_ref[0, 2880:4880, :] + s1_ref[0, 2880:4880, :]
    h = (s + g_ref[...]) * d + b_ref[0]
    h = jnp.maximum(h, 0.0)
    out_ref[...] = (
        jnp.dot(h, wout_ref[...], preferred_element_type=jnp.float32)
        + bout_ref[0]
    )


def _tc_final(s4, g4, dis16, b4, wout_pad, bout_pad):
    return pl.pallas_call(
        _tc_final_body,
        grid=(1,),
        in_specs=[
            pl.BlockSpec((1, ACC_H, D), lambda i: (2, 0, 0)),
            pl.BlockSpec((1, ACC_H, D), lambda i: (3, 0, 0)),
            pl.BlockSpec((N_OUT, D), lambda i: (4, 0)),
            pl.BlockSpec((N_OUT, 16), lambda i: (4, 0)),
            pl.BlockSpec((1, D), lambda i: (0, 0)),
            pl.BlockSpec((D, D), lambda i: (0, 0)),
            pl.BlockSpec((1, D), lambda i: (0, 0)),
        ],
        out_specs=pl.BlockSpec((N_OUT, D), lambda i: (0, 0)),
        out_shape=jax.ShapeDtypeStruct((N_OUT, D), jnp.float32),
    )(s4, s4, g4, dis16, b4, wout_pad, bout_pad)


@jax.jit
def _run(x, edge_index, W1, b1, W2, b2, W3, b3, W4, b4, Wout, bout):
    src = edge_index[0].astype(jnp.int32)
    dst = edge_index[1].astype(jnp.int32)
    npad = E_PAD - E
    src_p = jnp.concatenate([src, jnp.full((npad,), N, jnp.int32)])
    dst_p = jnp.concatenate([dst, jnp.full((npad,), N_PAD - 1, jnp.int32)])
    src_p, dst_p = jax.lax.sort((src_p, dst_p), num_keys=1)
    mask_a = dst_p < NH
    src2 = jnp.concatenate([
        jnp.where(mask_a, src_p, N),
        jnp.where(mask_a, N, src_p),
    ]).reshape(2 * E_PAD // CH, CH)
    dst2 = jnp.concatenate([
        jnp.where(mask_a, dst_p, JUNK),
        jnp.where(mask_a, JUNK, dst_p - NH),
    ]).reshape(2 * E_PAD // CH, CH)

    x_pad = jnp.pad(x, ((0, N_PAD - N), (0, 0)))
    zeros_big = jnp.zeros((RPT_A, D), jnp.float32)

    ones_mat = jnp.ones((N_PAD, D), jnp.float32)
    deg4 = _sc_scatter(ones_mat, jnp.zeros_like(src2), dst2, zeros_big)
    g, dis16 = _tc_layer1(x_pad, W1, deg4)

    b1r = b1.reshape(1, D)
    b2r = b2.reshape(1, D)
    b3r = b3.reshape(1, D)
    for w_next, b_prev in ((W2, b1r), (W3, b2r), (W4, b3r)):
        s4 = _sc_scatter(g, src2, dst2, zeros_big)
        g = _tc_mid(s4, g, dis16, b_prev, w_next)

    s4 = _sc_scatter(g, src2, dst2, zeros_big)
    wout_pad = jnp.pad(Wout, ((0, 0), (0, D - OUT_C)))
    bout_pad = jnp.pad(bout, (0, D - OUT_C)).reshape(1, D)
    outp = _tc_final(s4, g, dis16, b4.reshape(1, D), wout_pad, bout_pad)
    return outp[:, :OUT_C]


def kernel(x, edge_index, W1, b1, W2, b2, W3, b3, W4, b4, Wout, bout):
    return _run(x, edge_index, W1, b1, W2, b2, W3, b3, W4, b4, Wout, bout)


# speedup vs baseline: 1.4751x; 1.4751x over previous
import functools

import jax
import jax.numpy as jnp
from jax import lax
from jax.experimental import pallas as pl
from jax.experimental.pallas import tpu as pltpu
from jax.experimental.pallas import tpu_sc as plsc

N = 10000
D = 128
OUT_C = 12
N_OUT = 2000

N_PAD = 10240
NH = 5120
ACC_H = 5248
JUNK = NH
E = 320000
CH = 128
TILES = 32
E_PAD = 327680
EPT = E_PAD // TILES
NCH = EPT // CH
TPC = 16
RPT_A = ACC_H // TPC

_sc_cache = {}


def _build_sc_kernel():
    if _sc_cache:
        return _sc_cache["scat"]
    mesh = plsc.VectorSubcoreMesh(core_axis_name="c", subcore_axis_name="s")

    NBUF = 4
    ROUNDS = NCH // NBUF

    @functools.partial(
        pl.kernel,
        out_type=jax.ShapeDtypeStruct((4, ACC_H, D), jnp.float32),
        mesh=mesh,
        scratch_types=[
            pltpu.VMEM((NCH, CH), jnp.int32),
            pltpu.VMEM((NCH, CH), jnp.int32),
        ] + [pltpu.VMEM((CH, D), jnp.float32)] * NBUF + [
            pltpu.VMEM_SHARED((ACC_H, D), jnp.float32),
        ] + [pltpu.SemaphoreType.DMA] * (2 * NBUF),
    )
    def sc_scatter(g_hbm, src_hbm, dst_hbm, zeros_hbm, out_hbm,
                   src_v, dst_v, *rest):
        rows = rest[:NBUF]
        acc = rest[NBUF]
        sems = rest[NBUF + 1:]
        gsems = sems[:NBUF]
        ssems = sems[NBUF:]
        c = lax.axis_index("c")
        s = lax.axis_index("s")
        tile = c * TPC + s

        for q in (0, 1):
            base = q * (TILES * NCH) + tile * NCH
            pltpu.sync_copy(src_hbm.at[pl.ds(base, NCH)], src_v)
            pltpu.sync_copy(dst_hbm.at[pl.ds(base, NCH)], dst_v)
            pltpu.sync_copy(zeros_hbm,
                            acc.at[pl.ds(s * RPT_A, RPT_A)])
            plsc.subcore_barrier()

            def g_desc(k, b):
                return pltpu.make_async_copy(
                    g_hbm.at[src_v.at[k]], rows[b], gsems[b])

            def s_desc(k, b):
                return pltpu.make_async_copy(
                    rows[b], acc.at[dst_v.at[k]], ssems[b])

            for b in range(NBUF):
                g_desc(b, b).start()

            def round_body(r, carry):
                k0 = r * NBUF
                for b in range(NBUF):
                    g_desc(k0 + b, b).wait()
                    s_desc(k0 + b, b).start(add=True)
                for b in range(NBUF):
                    s_desc(k0 + b, b).wait()
                    g_desc(k0 + NBUF + b, b).start()
                return carry

            lax.fori_loop(0, ROUNDS - 1, round_body, 0)
            k0 = (ROUNDS - 1) * NBUF
            for b in range(NBUF):
                g_desc(k0 + b, b).wait()
                s_desc(k0 + b, b).start(add=True)
            for b in range(NBUF):
                s_desc(k0 + b, b).wait()
            plsc.subcore_barrier()
            pltpu.sync_copy(acc.at[pl.ds(s * RPT_A, RPT_A)],
                            out_hbm.at[2 * q + c].at[pl.ds(s * RPT_A, RPT_A)])

    _sc_cache["scat"] = sc_scatter
    return sc_scatter


def _sc_scatter(g, src2, dst2, zeros_big):
    return _build_sc_kernel()(g, src2, dst2, zeros_big)


_BLK = 1280
_GRID = N_PAD // _BLK


def _s_specs():
    return [
        pl.BlockSpec((1, _BLK, D), lambda i: (2 * (i // 4), i % 4, 0)),
        pl.BlockSpec((1, _BLK, D), lambda i: (2 * (i // 4) + 1, i % 4, 0)),
    ]


def _tc_layer1_body(x_ref, w_ref, d0_ref, d1_ref, g_ref, dis_ref):
    i = pl.program_id(0)
    rows = i * _BLK + lax.broadcasted_iota(jnp.int32, (_BLK, 1), 0)
    cnt = d0_ref[0, :, 0:1] + d1_ref[0, :, 0:1] + 1.0
    dis = lax.rsqrt(cnt) * (rows < N).astype(jnp.float32)
    dis_ref[...] = jnp.broadcast_to(dis, (_BLK, 16))
    h = jnp.dot(x_ref[...], w_ref[...], preferred_element_type=jnp.float32)
    g_ref[...] = h * dis


def _tc_layer1(x_pad, w1, deg4):
    return pl.pallas_call(
        _tc_layer1_body,
        grid=(_GRID,),
        in_specs=[
            pl.BlockSpec((_BLK, D), lambda i: (i, 0)),
            pl.BlockSpec((D, D), lambda i: (0, 0)),
        ] + _s_specs(),
        out_specs=[
            pl.BlockSpec((_BLK, D), lambda i: (i, 0)),
            pl.BlockSpec((_BLK, 16), lambda i: (i, 0)),
        ],
        out_shape=[
            jax.ShapeDtypeStruct((N_PAD, D), jnp.float32),
            jax.ShapeDtypeStruct((N_PAD, 16), jnp.float32),
        ],
    )(x_pad, w1, deg4, deg4)


def _tc_mid_body(s0_ref, s1_ref, g_ref, dis_ref, b_ref, w_ref, out_ref):
    d = dis_ref[:, 0:1]
    xk = (s0_ref[0] + s1_ref[0] + g_ref[...]) * d + b_ref[0]
    xk = jnp.maximum(xk, 0.0)
    h = jnp.dot(xk, w_ref[...], preferred_element_type=jnp.float32)
    out_ref[...] = h * d


def _tc_mid(s4, g_prev, dis16, b_prev, w_next):
    return pl.pallas_call(
        _tc_mid_body,
        grid=(_GRID,),
        in_specs=_s_specs() + [
            pl.BlockSpec((_BLK, D), lambda i: (i, 0)),
            pl.BlockSpec((_BLK, 16), lambda i: (i, 0)),
            pl.BlockSpec((1, D), lambda i: (0, 0)),
            pl.BlockSpec((D, D), lambda i: (0, 0)),
        ],
        out_specs=pl.BlockSpec((_BLK, D), lambda i: (i, 0)),
        out_shape=jax.ShapeDtypeStruct((N_PAD, D), jnp.float32),
    )(s4, s4, g_prev, dis16, b_prev, w_next)


def _tc_final_body(s0_ref, s1_ref, g_ref, dis_ref, b_ref, wout_ref, bout_ref,
                   out_ref):
    d = dis_ref[:, 0:1]
    s = s0_ref[0, 2880:4880, :] + s1_ref[0, 2880:4880, :]
    h = (s + g_ref[...]) * d + b_ref[0]
    h = jnp.maximum(h, 0.0)
    out_ref[...] = (
        jnp.dot(h, wout_ref[...], preferred_element_type=jnp.float32)
        + bout_ref[0]
    )


def _tc_final(s4, g4, dis16, b4, wout_pad, bout_pad):
    return pl.pallas_call(
        _tc_final_body,
        grid=(1,),
        in_specs=[
            pl.BlockSpec((1, ACC_H, D), lambda i: (2, 0, 0)),
            pl.BlockSpec((1, ACC_H, D), lambda i: (3, 0, 0)),
            pl.BlockSpec((N_OUT, D), lambda i: (4, 0)),
            pl.BlockSpec((N_OUT, 16), lambda i: (4, 0)),
            pl.BlockSpec((1, D), lambda i: (0, 0)),
            pl.BlockSpec((D, D), lambda i: (0, 0)),
            pl.BlockSpec((1, D), lambda i: (0, 0)),
        ],
        out_specs=pl.BlockSpec((N_OUT, D), lambda i: (0, 0)),
        out_shape=jax.ShapeDtypeStruct((N_OUT, D), jnp.float32),
    )(s4, s4, g4, dis16, b4, wout_pad, bout_pad)


@jax.jit
def _run(x, edge_index, W1, b1, W2, b2, W3, b3, W4, b4, Wout, bout):
    src = edge_index[0].astype(jnp.int32)
    dst = edge_index[1].astype(jnp.int32)
    npad = E_PAD - E
    src_p = jnp.concatenate([src, jnp.full((npad,), N, jnp.int32)])
    dst_p = jnp.concatenate([dst, jnp.full((npad,), N_PAD - 1, jnp.int32)])
    src_p, dst_p = jax.lax.sort((src_p, dst_p), num_keys=1)
    mask_a = dst_p < NH
    junk_dst = JUNK + (jnp.arange(E_PAD, dtype=jnp.int32) % (ACC_H - NH))
    src2 = jnp.concatenate([
        jnp.where(mask_a, src_p, N),
        jnp.where(mask_a, N, src_p),
    ]).reshape(2 * E_PAD // CH, CH)
    dst2 = jnp.concatenate([
        jnp.where(mask_a, dst_p, junk_dst),
        jnp.where(mask_a, junk_dst, dst_p - NH),
    ]).reshape(2 * E_PAD // CH, CH)

    x_pad = jnp.pad(x, ((0, N_PAD - N), (0, 0)))
    zeros_big = jnp.zeros((RPT_A, D), jnp.float32)

    ones_mat = jnp.ones((N_PAD, D), jnp.float32)
    deg_src2 = (jnp.arange(2 * E_PAD, dtype=jnp.int32) % N_PAD).reshape(
        2 * E_PAD // CH, CH)
    deg4 = _sc_scatter(ones_mat, deg_src2, dst2, zeros_big)
    g, dis16 = _tc_layer1(x_pad, W1, deg4)

    b1r = b1.reshape(1, D)
    b2r = b2.reshape(1, D)
    b3r = b3.reshape(1, D)
    for w_next, b_prev in ((W2, b1r), (W3, b2r), (W4, b3r)):
        s4 = _sc_scatter(g, src2, dst2, zeros_big)
        g = _tc_mid(s4, g, dis16, b_prev, w_next)

    s4 = _sc_scatter(g, src2, dst2, zeros_big)
    wout_pad = jnp.pad(Wout, ((0, 0), (0, D - OUT_C)))
    bout_pad = jnp.pad(bout, (0, D - OUT_C)).reshape(1, D)
    outp = _tc_final(s4, g, dis16, b4.reshape(1, D), wout_pad, bout_pad)
    return outp[:, :OUT_C]


def kernel(x, edge_index, W1, b1, W2, b2, W3, b3, W4, b4, Wout, bout):
    return _run(x, edge_index, W1, b1, W2, b2, W3, b3, W4, b4, Wout, bout)


# speedup vs baseline: 23.8105x; 16.1416x over previous
import functools

import jax
import jax.numpy as jnp
from jax import lax
from jax.experimental import pallas as pl
from jax.experimental.pallas import tpu as pltpu
from jax.experimental.pallas import tpu_sc as plsc

N = 10000
D = 128
OUT_C = 12
N_OUT = 2000

N_PAD = 10240
NH = 5120
ACC_H = 5248
JUNK = NH
E = 320000
CH = 128
TILES = 32
E_PAD = 327680
EPT = E_PAD // TILES
NCH = EPT // CH
TPC = 16
RPT_A = ACC_H // TPC

_sc_cache = {}


def _build_sc_kernel():
    if _sc_cache:
        return _sc_cache["scat"]
    mesh = plsc.VectorSubcoreMesh(core_axis_name="c", subcore_axis_name="s")

    NBUF = 4
    ROUNDS = NCH // NBUF

    @functools.partial(
        pl.kernel,
        out_type=jax.ShapeDtypeStruct((4, ACC_H, D), jnp.float32),
        mesh=mesh,
        scratch_types=[
            pltpu.VMEM((NCH, CH), jnp.int32),
            pltpu.VMEM((NCH, CH), jnp.int32),
        ] + [pltpu.VMEM((CH, D), jnp.float32)] * NBUF + [
            pltpu.VMEM_SHARED((ACC_H, D), jnp.float32),
        ] + [pltpu.SemaphoreType.DMA] * (2 * NBUF),
    )
    def sc_scatter(g_hbm, src_hbm, dst_hbm, zeros_hbm, out_hbm,
                   src_v, dst_v, *rest):
        rows = rest[:NBUF]
        acc = rest[NBUF]
        sems = rest[NBUF + 1:]
        gsems = sems[:NBUF]
        ssems = sems[NBUF:]
        c = lax.axis_index("c")
        s = lax.axis_index("s")
        tile = c * TPC + s

        for q in (0, 1):
            base = q * (TILES * NCH) + tile * NCH
            pltpu.sync_copy(src_hbm.at[pl.ds(base, NCH)], src_v)
            pltpu.sync_copy(dst_hbm.at[pl.ds(base, NCH)], dst_v)
            pltpu.sync_copy(zeros_hbm,
                            acc.at[pl.ds(s * RPT_A, RPT_A)])
            plsc.subcore_barrier()

            def g_desc(k, b):
                return pltpu.make_async_copy(
                    g_hbm.at[src_v.at[k]], rows[b], gsems[b])

            def s_desc(k, b):
                return pltpu.make_async_copy(
                    rows[b], acc.at[dst_v.at[k]], ssems[b])

            for b in range(NBUF):
                g_desc(b, b).start()

            def round_body(r, carry):
                k0 = r * NBUF
                for b in range(NBUF):
                    g_desc(k0 + b, b).wait()
                    s_desc(k0 + b, b).start(add=True)
                for b in range(NBUF):
                    s_desc(k0 + b, b).wait()
                    g_desc(k0 + NBUF + b, b).start()
                return carry

            lax.fori_loop(0, ROUNDS - 1, round_body, 0)
            k0 = (ROUNDS - 1) * NBUF
            for b in range(NBUF):
                g_desc(k0 + b, b).wait()
                s_desc(k0 + b, b).start(add=True)
            for b in range(NBUF):
                s_desc(k0 + b, b).wait()
            plsc.subcore_barrier()
            pltpu.sync_copy(acc.at[pl.ds(s * RPT_A, RPT_A)],
                            out_hbm.at[2 * q + c].at[pl.ds(s * RPT_A, RPT_A)])

    _sc_cache["scat"] = sc_scatter
    return sc_scatter


def _sc_scatter(g, src2, dst2, zeros_big):
    return _build_sc_kernel()(g, src2, dst2, zeros_big)


_BLK = 1280
_GRID = N_PAD // _BLK


def _s_specs():
    return [
        pl.BlockSpec((1, _BLK, D), lambda i: (2 * (i // 4), i % 4, 0)),
        pl.BlockSpec((1, _BLK, D), lambda i: (2 * (i // 4) + 1, i % 4, 0)),
    ]


def _tc_layer1_body(x_ref, w_ref, d0_ref, d1_ref, g_ref, dis_ref):
    i = pl.program_id(0)
    rows = i * _BLK + lax.broadcasted_iota(jnp.int32, (_BLK, 1), 0)
    cnt = d0_ref[0, :, 0:1] + d1_ref[0, :, 0:1] + 1.0
    dis = lax.rsqrt(cnt) * (rows < N).astype(jnp.float32)
    dis_ref[...] = jnp.broadcast_to(dis, (_BLK, 16))
    h = jnp.dot(x_ref[...], w_ref[...], preferred_element_type=jnp.float32)
    g_ref[...] = h * dis


def _tc_layer1(x_pad, w1, deg4):
    return pl.pallas_call(
        _tc_layer1_body,
        grid=(_GRID,),
        in_specs=[
            pl.BlockSpec((_BLK, D), lambda i: (i, 0)),
            pl.BlockSpec((D, D), lambda i: (0, 0)),
        ] + _s_specs(),
        out_specs=[
            pl.BlockSpec((_BLK, D), lambda i: (i, 0)),
            pl.BlockSpec((_BLK, 16), lambda i: (i, 0)),
        ],
        out_shape=[
            jax.ShapeDtypeStruct((N_PAD, D), jnp.float32),
            jax.ShapeDtypeStruct((N_PAD, 16), jnp.float32),
        ],
    )(x_pad, w1, deg4, deg4)


def _tc_mid_body(s0_ref, s1_ref, g_ref, dis_ref, b_ref, w_ref, out_ref):
    d = dis_ref[:, 0:1]
    xk = (s0_ref[0] + s1_ref[0] + g_ref[...]) * d + b_ref[0]
    xk = jnp.maximum(xk, 0.0)
    h = jnp.dot(xk, w_ref[...], preferred_element_type=jnp.float32)
    out_ref[...] = h * d


def _tc_mid(s4, g_prev, dis16, b_prev, w_next):
    return pl.pallas_call(
        _tc_mid_body,
        grid=(_GRID,),
        in_specs=_s_specs() + [
            pl.BlockSpec((_BLK, D), lambda i: (i, 0)),
            pl.BlockSpec((_BLK, 16), lambda i: (i, 0)),
            pl.BlockSpec((1, D), lambda i: (0, 0)),
            pl.BlockSpec((D, D), lambda i: (0, 0)),
        ],
        out_specs=pl.BlockSpec((_BLK, D), lambda i: (i, 0)),
        out_shape=jax.ShapeDtypeStruct((N_PAD, D), jnp.float32),
    )(s4, s4, g_prev, dis16, b_prev, w_next)


def _tc_final_body(s0_ref, s1_ref, g_ref, dis_ref, b_ref, wout_ref, bout_ref,
                   out_ref):
    d = dis_ref[:, 0:1]
    s = s0_ref[0, 2880:4880, :] + s1_ref[0, 2880:4880, :]
    h = (s + g_ref[...]) * d + b_ref[0]
    h = jnp.maximum(h, 0.0)
    out_ref[...] = (
        jnp.dot(h, wout_ref[...], preferred_element_type=jnp.float32)
        + bout_ref[0]
    )


def _tc_final(s4, g4, dis16, b4, wout_pad, bout_pad):
    return pl.pallas_call(
        _tc_final_body,
        grid=(1,),
        in_specs=[
            pl.BlockSpec((1, ACC_H, D), lambda i: (2, 0, 0)),
            pl.BlockSpec((1, ACC_H, D), lambda i: (3, 0, 0)),
            pl.BlockSpec((N_OUT, D), lambda i: (4, 0)),
            pl.BlockSpec((N_OUT, 16), lambda i: (4, 0)),
            pl.BlockSpec((1, D), lambda i: (0, 0)),
            pl.BlockSpec((D, D), lambda i: (0, 0)),
            pl.BlockSpec((1, D), lambda i: (0, 0)),
        ],
        out_specs=pl.BlockSpec((N_OUT, D), lambda i: (0, 0)),
        out_shape=jax.ShapeDtypeStruct((N_OUT, D), jnp.float32),
    )(s4, s4, g4, dis16, b4, wout_pad, bout_pad)


@jax.jit
def _run(x, edge_index, W1, b1, W2, b2, W3, b3, W4, b4, Wout, bout):
    src = edge_index[0].astype(jnp.int32)
    dst = edge_index[1].astype(jnp.int32)
    npad = E_PAD - E
    src_p = jnp.concatenate([src, jnp.full((npad,), N, jnp.int32)])
    dst_p = jnp.concatenate([dst, jnp.full((npad,), N_PAD - 1, jnp.int32)])
    src_p, dst_p = jax.lax.sort((src_p, dst_p), num_keys=1)
    mask_a = dst_p < NH
    junk_dst = JUNK + (jnp.arange(E_PAD, dtype=jnp.int32) % (ACC_H - NH))
    pad_src = N + (jnp.arange(E_PAD, dtype=jnp.int32) % (N_PAD - N))
    src2 = jnp.concatenate([
        jnp.where(mask_a, src_p, pad_src),
        jnp.where(mask_a, pad_src, src_p),
    ]).reshape(2 * E_PAD // CH, CH)
    dst2 = jnp.concatenate([
        jnp.where(mask_a, dst_p, junk_dst),
        jnp.where(mask_a, junk_dst, dst_p - NH),
    ]).reshape(2 * E_PAD // CH, CH)

    x_pad = jnp.pad(x, ((0, N_PAD - N), (0, 0)))
    zeros_big = jnp.zeros((RPT_A, D), jnp.float32)

    ones_mat = jnp.ones((N_PAD, D), jnp.float32)
    deg_src2 = (jnp.arange(2 * E_PAD, dtype=jnp.int32) % N_PAD).reshape(
        2 * E_PAD // CH, CH)
    deg4 = _sc_scatter(ones_mat, deg_src2, dst2, zeros_big)
    g, dis16 = _tc_layer1(x_pad, W1, deg4)

    b1r = b1.reshape(1, D)
    b2r = b2.reshape(1, D)
    b3r = b3.reshape(1, D)
    for w_next, b_prev in ((W2, b1r), (W3, b2r), (W4, b3r)):
        s4 = _sc_scatter(g, src2, dst2, zeros_big)
        g = _tc_mid(s4, g, dis16, b_prev, w_next)

    s4 = _sc_scatter(g, src2, dst2, zeros_big)
    wout_pad = jnp.pad(Wout, ((0, 0), (0, D - OUT_C)))
    bout_pad = jnp.pad(bout, (0, D - OUT_C)).reshape(1, D)
    outp = _tc_final(s4, g, dis16, b4.reshape(1, D), wout_pad, bout_pad)
    return outp[:, :OUT_C]


def kernel(x, edge_index, W1, b1, W2, b2, W3, b3, W4, b4, Wout, bout):
    return _run(x, edge_index, W1, b1, W2, b2, W3, b3, W4, b4, Wout, bout)
